# Initial kernel scaffold; baseline (speedup 1.0000x reference)
import jax
import jax.numpy as jnp
from jax.experimental import pallas as pl


def kernel(z_x):
    raise NotImplementedError("write your pallas kernel here")


# speedup vs baseline: 129.2823x; 129.2823x over previous
import jax
import jax.numpy as jnp
from jax import lax
from jax.experimental import pallas as pl

_N = 8192
_D = 64
_BLOCK = 256


def _knn_block_kernel(xb_ref, x_ref, out_ref):
    i = pl.program_id(0)
    x = x_ref[...]
    norms = jnp.sqrt(jnp.sum(x * x, axis=1, keepdims=True))
    zn = x / jnp.maximum(norms, 1e-12)
    xb = xb_ref[...]
    bnorms = jnp.sqrt(jnp.sum(xb * xb, axis=1, keepdims=True))
    zb = xb / jnp.maximum(bnorms, 1e-12)
    s = lax.dot_general(zb, zn, (((1,), (1,)), ((), ())),
                        preferred_element_type=jnp.float32)
    col = lax.broadcasted_iota(jnp.int32, (_BLOCK, _N), 1)
    row = lax.broadcasted_iota(jnp.int32, (_BLOCK, _N), 0) + i * _BLOCK
    s = jnp.where(col == row, 0.0, s)
    v1 = jnp.max(s, axis=1, keepdims=True)
    idx1 = jnp.min(jnp.where(s == v1, col, _N), axis=1, keepdims=True)
    v2 = jnp.max(jnp.where(col == idx1, -jnp.inf, s), axis=1, keepdims=True)
    out_ref[...] = jnp.where(s >= v2, s, 0.0)


def kernel(z_x):
    return pl.pallas_call(
        _knn_block_kernel,
        grid=(_N // _BLOCK,),
        in_specs=[pl.BlockSpec((_BLOCK, _D), lambda i: (i, 0)),
                  pl.BlockSpec((_N, _D), lambda i: (0, 0))],
        out_specs=pl.BlockSpec((_BLOCK, _N), lambda i: (i, 0)),
        out_shape=jax.ShapeDtypeStruct((_N, _N), jnp.float32),
    )(z_x, z_x)


# speedup vs baseline: 177.1951x; 1.3706x over previous
import jax
import jax.numpy as jnp
from jax import lax
from jax.experimental import pallas as pl
from jax.experimental.pallas import tpu as pltpu

_N = 8192
_D = 64
_BLOCK = 256
_NEG = float("-inf")


def _knn_block_kernel(xb_ref, x_ref, out_ref, zn_ref):
    i = pl.program_id(0)

    @pl.when(i == 0)
    def _():
        x = x_ref[...]
        norms = jnp.sqrt(jnp.sum(x * x, axis=1, keepdims=True))
        zn_ref[...] = x / jnp.maximum(norms, 1e-12)

    xb = xb_ref[...]
    bnorms = jnp.sqrt(jnp.sum(xb * xb, axis=1, keepdims=True))
    zb = xb / jnp.maximum(bnorms, 1e-12)
    s = lax.dot_general(zb, zn_ref[...], (((1,), (1,)), ((), ())),
                        preferred_element_type=jnp.float32)
    off = s < jnp.max(s, axis=1, keepdims=True)
    v1 = jnp.max(jnp.where(off, s, _NEG), axis=1, keepdims=True)
    v2 = jnp.max(jnp.where(s < v1, s, _NEG), axis=1, keepdims=True)
    out_ref[...] = jnp.where((s >= v2) & off, s, 0.0)


def kernel(z_x):
    return pl.pallas_call(
        _knn_block_kernel,
        grid=(_N // _BLOCK,),
        in_specs=[pl.BlockSpec((_BLOCK, _D), lambda i: (i, 0)),
                  pl.BlockSpec((_N, _D), lambda i: (0, 0))],
        out_specs=pl.BlockSpec((_BLOCK, _N), lambda i: (i, 0)),
        out_shape=jax.ShapeDtypeStruct((_N, _N), jnp.float32),
        scratch_shapes=[pltpu.VMEM((_N, _D), jnp.float32)],
    )(z_x, z_x)


# speedup vs baseline: 212.0599x; 1.1968x over previous
import jax
import jax.numpy as jnp
from jax import lax
from jax.experimental import pallas as pl

_N = 8192
_D = 64
_BLOCK = 256
_NEG = float("-inf")


def _normalize_kernel(x_ref, zn_ref):
    x = x_ref[...]
    norms = jnp.sqrt(jnp.sum(x * x, axis=1, keepdims=True))
    zn_ref[...] = x / jnp.maximum(norms, 1e-12)


def _knn_block_kernel(zb_ref, zn_ref, out_ref):
    i = pl.program_id(0)
    s = lax.dot_general(zb_ref[...], zn_ref[...], (((1,), (1,)), ((), ())),
                        preferred_element_type=jnp.float32)
    col = lax.broadcasted_iota(jnp.int32, (_BLOCK, _N), 1)
    row = lax.broadcasted_iota(jnp.int32, (_BLOCK, _N), 0) + i * _BLOCK
    sm = jnp.where(col == row, _NEG, s)
    v1 = jnp.max(sm, axis=1, keepdims=True)
    v2 = jnp.max(jnp.where(sm < v1, sm, _NEG), axis=1, keepdims=True)
    out_ref[...] = jnp.where(sm >= v2, sm, 0.0)


def kernel(z_x):
    zn = pl.pallas_call(
        _normalize_kernel,
        out_shape=jax.ShapeDtypeStruct((_N, _D), jnp.float32),
    )(z_x)
    return pl.pallas_call(
        _knn_block_kernel,
        grid=(_N // _BLOCK,),
        in_specs=[pl.BlockSpec((_BLOCK, _D), lambda i: (i, 0)),
                  pl.BlockSpec((_N, _D), lambda i: (0, 0))],
        out_specs=pl.BlockSpec((_BLOCK, _N), lambda i: (i, 0)),
        out_shape=jax.ShapeDtypeStruct((_N, _N), jnp.float32),
    )(zn, zn)


# speedup vs baseline: 234.0943x; 1.1039x over previous
import jax
import jax.numpy as jnp
from jax import lax
from jax.experimental import pallas as pl

_N = 8192
_D = 64
_BLOCK = 256
_NEG = float("-inf")
_G = 8
_C = 128


def _normalize_kernel(x_ref, zn_ref):
    x = x_ref[...]
    norms = jnp.sqrt(jnp.sum(x * x, axis=1, keepdims=True))
    zn_ref[...] = x / jnp.maximum(norms, 1e-12)


def _knn_block_kernel(zb_ref, zn_ref, out_ref):
    i = pl.program_id(0)
    s = lax.dot_general(zb_ref[...], zn_ref[...], (((1,), (1,)), ((), ())),
                        preferred_element_type=jnp.float32)
    col = lax.broadcasted_iota(jnp.int32, (_BLOCK, _N), 1)
    row = lax.broadcasted_iota(jnp.int32, (_BLOCK, _N), 0) + i * _BLOCK
    sm = jnp.where(col == row, _NEG, s)

    for g in range(_BLOCK // _G):
        smg = sm[g * _G:(g + 1) * _G, :]
        a = smg[:, 0:_C]
        b = jnp.full((_G, _C), _NEG, jnp.float32)
        for k in range(1, _N // _C):
            x = smg[:, k * _C:(k + 1) * _C]
            t = jnp.minimum(a, x)
            a = jnp.maximum(a, x)
            b = jnp.maximum(b, t)
        v1 = jnp.max(a, axis=1, keepdims=True)
        eq = a == v1
        l2 = jnp.max(jnp.where(eq, _NEG, a), axis=1, keepdims=True)
        bat = jnp.max(jnp.where(eq, b, _NEG), axis=1, keepdims=True)
        v2 = jnp.maximum(l2, bat)
        out_ref[g * _G:(g + 1) * _G, :] = jnp.where(smg >= v2, smg, 0.0)


def kernel(z_x):
    zn = pl.pallas_call(
        _normalize_kernel,
        out_shape=jax.ShapeDtypeStruct((_N, _D), jnp.float32),
    )(z_x)
    return pl.pallas_call(
        _knn_block_kernel,
        grid=(_N // _BLOCK,),
        in_specs=[pl.BlockSpec((_BLOCK, _D), lambda i: (i, 0)),
                  pl.BlockSpec((_N, _D), lambda i: (0, 0))],
        out_specs=pl.BlockSpec((_BLOCK, _N), lambda i: (i, 0)),
        out_shape=jax.ShapeDtypeStruct((_N, _N), jnp.float32),
    )(zn, zn)
